# baseline (speedup 1.0000x reference)
import jax
import jax.numpy as jnp
from jax.experimental import pallas as pl
from jax.experimental.pallas import tpu as pltpu

_B, _N, _F = 16, 4096, 512
_E = 64
_CPB = 4
_ROWS = _N // _CPB
_NCHUNK = _B * _CPB
_NBUF = 8
_LA = 4
_WD = 4


def _unpool_kernel(a_ref, o_ref, bufs_ref, lo_ref, hi_ref, tail_ref,
                   in_sems, out_sems, lo_sem, hi_sem, tail_sem):
    def in_cp(c):
        b, q = divmod(c, _CPB)
        return pltpu.make_async_copy(
            a_ref.at[b, pl.ds(q * _ROWS, _ROWS), :],
            bufs_ref.at[c % _NBUF], in_sems.at[c % _NBUF])

    def out_cp(c):
        b, q = divmod(c, _CPB)
        return pltpu.make_async_copy(
            bufs_ref.at[c % _NBUF],
            o_ref.at[b, pl.ds(q * _ROWS, _ROWS), :], out_sems.at[c % _NBUF])

    lo_cp = pltpu.make_async_copy(a_ref.at[:, 0:_E, :], lo_ref, lo_sem)
    hi_cp = pltpu.make_async_copy(a_ref.at[:, 2048:2048 + _E, :], hi_ref, hi_sem)
    lo_cp.start()
    hi_cp.start()
    for c in range(_LA - 1):
        in_cp(c).start()

    lo_cp.wait()
    hi_cp.wait()
    tail_ref[...] = 0.5 * (lo_ref[...] + hi_ref[...])
    tail_cp = pltpu.make_async_copy(tail_ref, o_ref.at[:, _N:_N + _E, :], tail_sem)
    tail_cp.start()

    for c in range(_NCHUNK):
        if c >= _WD:
            out_cp(c - _WD).wait()
        if c + _LA - 1 < _NCHUNK:
            in_cp(c + _LA - 1).start()
        in_cp(c).wait()
        out_cp(c).start()
    for c in range(_NCHUNK - _WD, _NCHUNK):
        out_cp(c).wait()
    tail_cp.wait()


def kernel(inputs):
    return pl.pallas_call(
        _unpool_kernel,
        in_specs=[pl.BlockSpec(memory_space=pl.ANY)],
        out_specs=pl.BlockSpec(memory_space=pl.ANY),
        out_shape=jax.ShapeDtypeStruct((_B, _N + _E, _F), inputs.dtype),
        scratch_shapes=[
            pltpu.VMEM((_NBUF, _ROWS, _F), inputs.dtype),
            pltpu.VMEM((_B, _E, _F), inputs.dtype),
            pltpu.VMEM((_B, _E, _F), inputs.dtype),
            pltpu.VMEM((_B, _E, _F), inputs.dtype),
            pltpu.SemaphoreType.DMA((_NBUF,)),
            pltpu.SemaphoreType.DMA((_NBUF,)),
            pltpu.SemaphoreType.DMA,
            pltpu.SemaphoreType.DMA,
            pltpu.SemaphoreType.DMA,
        ],
    )(inputs)


# speedup vs baseline: 1.0060x; 1.0060x over previous
import jax
import jax.numpy as jnp
from jax.experimental import pallas as pl
from jax.experimental.pallas import tpu as pltpu

_B, _N, _F = 16, 4096, 512
_E = 64
_CPB = 2
_ROWS = _N // _CPB
_NCHUNK = _B * _CPB
_NBUF = 6
_LA = 3
_WD = 3


def _unpool_kernel(a_ref, o_ref, bufs_ref, lo_ref, hi_ref, tail_ref,
                   in_sems, out_sems, lo_sem, hi_sem, tail_sem):
    def in_cp(c):
        b, q = divmod(c, _CPB)
        return pltpu.make_async_copy(
            a_ref.at[b, pl.ds(q * _ROWS, _ROWS), :],
            bufs_ref.at[c % _NBUF], in_sems.at[c % _NBUF])

    def out_cp(c):
        b, q = divmod(c, _CPB)
        return pltpu.make_async_copy(
            bufs_ref.at[c % _NBUF],
            o_ref.at[b, pl.ds(q * _ROWS, _ROWS), :], out_sems.at[c % _NBUF])

    lo_cp = pltpu.make_async_copy(a_ref.at[:, 0:_E, :], lo_ref, lo_sem)
    hi_cp = pltpu.make_async_copy(a_ref.at[:, 2048:2048 + _E, :], hi_ref, hi_sem)
    lo_cp.start()
    hi_cp.start()
    for c in range(_LA - 1):
        in_cp(c).start()

    lo_cp.wait()
    hi_cp.wait()
    tail_ref[...] = 0.5 * (lo_ref[...] + hi_ref[...])
    tail_cp = pltpu.make_async_copy(tail_ref, o_ref.at[:, _N:_N + _E, :], tail_sem)
    tail_cp.start()

    for c in range(_NCHUNK):
        if c >= _WD:
            out_cp(c - _WD).wait()
        if c + _LA - 1 < _NCHUNK:
            in_cp(c + _LA - 1).start()
        in_cp(c).wait()
        out_cp(c).start()
    for c in range(_NCHUNK - _WD, _NCHUNK):
        out_cp(c).wait()
    tail_cp.wait()


def kernel(inputs):
    return pl.pallas_call(
        _unpool_kernel,
        in_specs=[pl.BlockSpec(memory_space=pl.ANY)],
        out_specs=pl.BlockSpec(memory_space=pl.ANY),
        out_shape=jax.ShapeDtypeStruct((_B, _N + _E, _F), inputs.dtype),
        scratch_shapes=[
            pltpu.VMEM((_NBUF, _ROWS, _F), inputs.dtype),
            pltpu.VMEM((_B, _E, _F), inputs.dtype),
            pltpu.VMEM((_B, _E, _F), inputs.dtype),
            pltpu.VMEM((_B, _E, _F), inputs.dtype),
            pltpu.SemaphoreType.DMA((_NBUF,)),
            pltpu.SemaphoreType.DMA((_NBUF,)),
            pltpu.SemaphoreType.DMA,
            pltpu.SemaphoreType.DMA,
            pltpu.SemaphoreType.DMA,
        ],
    )(inputs)


# speedup vs baseline: 1.0179x; 1.0118x over previous
import jax
import jax.numpy as jnp
from jax.experimental import pallas as pl
from jax.experimental.pallas import tpu as pltpu

_B, _N, _F = 16, 4096, 512
_E = 64
_CPB = 1
_ROWS = _N // _CPB
_NCHUNK = _B * _CPB
_NBUF = 4
_LA = 3
_WD = 2


def _unpool_kernel(a_ref, o_ref, bufs_ref, lo_ref, hi_ref, tail_ref,
                   in_sems, out_sems, lo_sem, hi_sem, tail_sem):
    def in_cp(c):
        b, q = divmod(c, _CPB)
        return pltpu.make_async_copy(
            a_ref.at[b, pl.ds(q * _ROWS, _ROWS), :],
            bufs_ref.at[c % _NBUF], in_sems.at[c % _NBUF])

    def out_cp(c):
        b, q = divmod(c, _CPB)
        return pltpu.make_async_copy(
            bufs_ref.at[c % _NBUF],
            o_ref.at[b, pl.ds(q * _ROWS, _ROWS), :], out_sems.at[c % _NBUF])

    lo_cp = pltpu.make_async_copy(a_ref.at[:, 0:_E, :], lo_ref, lo_sem)
    hi_cp = pltpu.make_async_copy(a_ref.at[:, 2048:2048 + _E, :], hi_ref, hi_sem)
    lo_cp.start()
    hi_cp.start()
    for c in range(_LA - 1):
        in_cp(c).start()

    lo_cp.wait()
    hi_cp.wait()
    tail_ref[...] = 0.5 * (lo_ref[...] + hi_ref[...])
    tail_cp = pltpu.make_async_copy(tail_ref, o_ref.at[:, _N:_N + _E, :], tail_sem)
    tail_cp.start()

    for c in range(_NCHUNK):
        if c >= _WD:
            out_cp(c - _WD).wait()
        if c + _LA - 1 < _NCHUNK:
            in_cp(c + _LA - 1).start()
        in_cp(c).wait()
        out_cp(c).start()
    for c in range(_NCHUNK - _WD, _NCHUNK):
        out_cp(c).wait()
    tail_cp.wait()


def kernel(inputs):
    return pl.pallas_call(
        _unpool_kernel,
        in_specs=[pl.BlockSpec(memory_space=pl.ANY)],
        out_specs=pl.BlockSpec(memory_space=pl.ANY),
        out_shape=jax.ShapeDtypeStruct((_B, _N + _E, _F), inputs.dtype),
        scratch_shapes=[
            pltpu.VMEM((_NBUF, _ROWS, _F), inputs.dtype),
            pltpu.VMEM((_B, _E, _F), inputs.dtype),
            pltpu.VMEM((_B, _E, _F), inputs.dtype),
            pltpu.VMEM((_B, _E, _F), inputs.dtype),
            pltpu.SemaphoreType.DMA((_NBUF,)),
            pltpu.SemaphoreType.DMA((_NBUF,)),
            pltpu.SemaphoreType.DMA,
            pltpu.SemaphoreType.DMA,
            pltpu.SemaphoreType.DMA,
        ],
    )(inputs)
